# Initial kernel scaffold; baseline (speedup 1.0000x reference)
import jax
import jax.numpy as jnp
from jax.experimental import pallas as pl


def kernel(x_fact, x_company, edge_attr_fc, edge_attr_cf, params, src_fc, dst_fc, src_cf, dst_cf, fact_batch, company_batch):
    raise NotImplementedError("write your pallas kernel here")


# speedup vs baseline: 7.0483x; 7.0483x over previous
import functools

import jax
import jax.numpy as jnp
from jax.experimental import pallas as pl
from jax.experimental.pallas import tpu as pltpu

NF = 10000
NC = 2000
E = 150000
H = 512
B = 64
DF = 512
DC = 128

NFP = 10240
NCP = 2048
EP = 151552
ER = EP // 128


def _gate_idx_kernel(a0_ref, a1_ref, src_ref, dst_ref, mix_ref,
                     w_ref, idx_ref):
    m0 = mix_ref[0, 0]
    m1 = mix_ref[0, 1]
    b0 = mix_ref[0, 2]
    stride = mix_ref[0, 3].astype(jnp.int32)
    z = a0_ref[:] * m0 + a1_ref[:] * m1 + b0
    w_ref[:] = jax.nn.sigmoid(z)
    idx_ref[:] = dst_ref[:] * stride + src_ref[:]


def _gate_idx(a0, a1, src, dst, mixer_W, mixer_b, stride):
    mix = jnp.concatenate([mixer_W[0], mixer_W[1], mixer_b,
                           jnp.full((1,), float(stride), jnp.float32)])
    mix = mix.reshape(1, 4)
    return pl.pallas_call(
        _gate_idx_kernel,
        out_shape=(jax.ShapeDtypeStruct((ER, 128), jnp.float32),
                   jax.ShapeDtypeStruct((ER, 128), jnp.int32)),
    )(a0, a1, src, dst, mix)


def _ln(x, g, b, eps=1e-5):
    m = jnp.mean(x, axis=-1, keepdims=True)
    v = jnp.mean((x - m) * (x - m), axis=-1, keepdims=True)
    return (x - m) / jnp.sqrt(v + eps) * g + b


def _encoder_kernel(x_ref, w_ref, b_ref, g_ref, beta_ref, o_ref):
    h = jnp.maximum(
        jnp.dot(x_ref[:], w_ref[:], preferred_element_type=jnp.float32)
        + b_ref[:], 0.0)
    o_ref[:] = _ln(h, g_ref[:], beta_ref[:])


def _encoder(x, w, b, g, beta, blk):
    n, d = x.shape
    grid = n // blk
    return pl.pallas_call(
        _encoder_kernel,
        grid=(grid,),
        in_specs=[
            pl.BlockSpec((blk, d), lambda i: (i, 0)),
            pl.BlockSpec((d, H), lambda i: (0, 0)),
            pl.BlockSpec((1, H), lambda i: (0, 0)),
            pl.BlockSpec((1, H), lambda i: (0, 0)),
            pl.BlockSpec((1, H), lambda i: (0, 0)),
        ],
        out_specs=pl.BlockSpec((blk, H), lambda i: (i, 0)),
        out_shape=jax.ShapeDtypeStruct((n, H), jnp.float32),
    )(x, w, b, g, beta)


def _conv_c_kernel(a_ref, xf_ref, xc_ref, relw_ref, relb_ref, rootw_ref,
                   g_ref, beta_ref, o_ref, acc_ref):
    k = pl.program_id(0)
    nk = pl.num_programs(0)

    @pl.when(k == 0)
    def _():
        acc_ref[:] = jnp.zeros_like(acc_ref)

    acc_ref[:] += jnp.dot(a_ref[:], xf_ref[:],
                          preferred_element_type=jnp.float32)

    @pl.when(k == nk - 1)
    def _():
        out = (jnp.dot(acc_ref[:], relw_ref[:],
                       preferred_element_type=jnp.float32)
               + relb_ref[:]
               + jnp.dot(xc_ref[:], rootw_ref[:],
                         preferred_element_type=jnp.float32))
        o_ref[:] = _ln(jnp.maximum(out, 0.0), g_ref[:], beta_ref[:])


def _conv_c(a_fc, xf, xc, relw, relb, rootw, g, beta):
    kb = 1024
    grid = NFP // kb
    return pl.pallas_call(
        _conv_c_kernel,
        grid=(grid,),
        in_specs=[
            pl.BlockSpec((NCP, kb), lambda k: (0, k)),
            pl.BlockSpec((kb, H), lambda k: (k, 0)),
            pl.BlockSpec((NCP, H), lambda k: (0, 0)),
            pl.BlockSpec((H, H), lambda k: (0, 0)),
            pl.BlockSpec((1, H), lambda k: (0, 0)),
            pl.BlockSpec((H, H), lambda k: (0, 0)),
            pl.BlockSpec((1, H), lambda k: (0, 0)),
            pl.BlockSpec((1, H), lambda k: (0, 0)),
        ],
        out_specs=pl.BlockSpec((NCP, H), lambda k: (0, 0)),
        out_shape=jax.ShapeDtypeStruct((NCP, H), jnp.float32),
        scratch_shapes=[pltpu.VMEM((NCP, H), jnp.float32)],
    )(a_fc, xf, xc, relw, relb, rootw, g, beta)


def _conv_f_kernel(a_ref, xc_ref, xf_ref, relw_ref, relb_ref, rootw_ref,
                   g_ref, beta_ref, o_ref):
    agg = jnp.dot(a_ref[:], xc_ref[:], preferred_element_type=jnp.float32)
    out = (jnp.dot(agg, relw_ref[:], preferred_element_type=jnp.float32)
           + relb_ref[:]
           + jnp.dot(xf_ref[:], rootw_ref[:],
                     preferred_element_type=jnp.float32))
    o_ref[:] = _ln(jnp.maximum(out, 0.0), g_ref[:], beta_ref[:])


def _conv_f(a_cf, xc, xf, relw, relb, rootw, g, beta):
    mb = 1024
    grid = NFP // mb
    return pl.pallas_call(
        _conv_f_kernel,
        grid=(grid,),
        in_specs=[
            pl.BlockSpec((mb, NCP), lambda m: (m, 0)),
            pl.BlockSpec((NCP, H), lambda m: (0, 0)),
            pl.BlockSpec((mb, H), lambda m: (m, 0)),
            pl.BlockSpec((H, H), lambda m: (0, 0)),
            pl.BlockSpec((1, H), lambda m: (0, 0)),
            pl.BlockSpec((H, H), lambda m: (0, 0)),
            pl.BlockSpec((1, H), lambda m: (0, 0)),
            pl.BlockSpec((1, H), lambda m: (0, 0)),
        ],
        out_specs=pl.BlockSpec((mb, H), lambda m: (m, 0)),
        out_shape=jax.ShapeDtypeStruct((NFP, H), jnp.float32),
    )(a_cf, xc, xf, relw, relb, rootw, g, beta)


def _pool_kernel(x_ref, batch_ref, sum_ref, cnt_ref, psum_ref, pcnt_ref):
    i = pl.program_id(0)
    n = pl.num_programs(0)

    @pl.when(i == 0)
    def _():
        psum_ref[:] = jnp.zeros_like(psum_ref)
        pcnt_ref[:] = jnp.zeros_like(pcnt_ref)

    b = batch_ref[0, :, :]
    mask = (jax.lax.broadcasted_iota(jnp.int32, (B, b.shape[1]), 0)
            == b).astype(jnp.float32)
    psum_ref[:] += jnp.dot(mask, x_ref[:], preferred_element_type=jnp.float32)
    pcnt_ref[:] += jnp.sum(mask, axis=1, keepdims=True)

    @pl.when(i == n - 1)
    def _():
        sum_ref[:] = psum_ref[:]
        cnt_ref[:] = pcnt_ref[:]


def _pool(x, batch3d, blk):
    n = x.shape[0]
    grid = n // blk
    return pl.pallas_call(
        _pool_kernel,
        grid=(grid,),
        in_specs=[
            pl.BlockSpec((blk, H), lambda i: (i, 0)),
            pl.BlockSpec((1, 1, blk), lambda i: (i, 0, 0)),
        ],
        out_specs=(pl.BlockSpec((B, H), lambda i: (0, 0)),
                   pl.BlockSpec((B, 1), lambda i: (0, 0))),
        out_shape=(jax.ShapeDtypeStruct((B, H), jnp.float32),
                   jax.ShapeDtypeStruct((B, 1), jnp.float32)),
        scratch_shapes=[pltpu.VMEM((B, H), jnp.float32),
                        pltpu.VMEM((B, 1), jnp.float32)],
    )(x, batch3d)


def _readout_kernel(fs_ref, fc_ref, cs_ref, cc_ref, gwf_ref, gwc_ref,
                    gb_ref, cw_ref, cb_ref, o_ref):
    fp = fs_ref[:] / jnp.maximum(fc_ref[:], 1.0)
    cp = cs_ref[:] / jnp.maximum(cc_ref[:], 1.0)
    z = (jnp.dot(fp, gwf_ref[:], preferred_element_type=jnp.float32)
         + jnp.dot(cp, gwc_ref[:], preferred_element_type=jnp.float32)
         + gb_ref[0, 0])
    alpha = jax.nn.sigmoid(z)
    pooled = alpha * fp + (1.0 - alpha) * cp
    o_ref[:] = (jnp.dot(pooled, cw_ref[:], preferred_element_type=jnp.float32)
                + cb_ref[0, 0])


def _readout(fs, fc, cs, cc, gwf, gwc, gb, cw, cb):
    return pl.pallas_call(
        _readout_kernel,
        out_shape=jax.ShapeDtypeStruct((B, 1), jnp.float32),
    )(fs, fc, cs, cc, gwf, gwc, gb, cw, cb)


def _pad_edges_f32(x):
    return jnp.pad(x, (0, EP - E)).reshape(ER, 128)


def _pad_edges_i32(x, fill):
    return jnp.pad(x.astype(jnp.int32), (0, EP - E),
                   constant_values=fill).reshape(ER, 128)


def _build_adj(idx, w, rows, cols):
    flat = jnp.zeros((rows * cols + 1,), jnp.float32)
    safe = jnp.where(idx < 0, rows * cols, idx)
    flat = flat.at[safe.reshape(-1)].add(w.reshape(-1))
    return flat[:-1].reshape(rows, cols)


def kernel(x_fact, x_company, edge_attr_fc, edge_attr_cf, params,
           src_fc, dst_fc, src_cf, dst_cf, fact_batch, company_batch):
    p = params

    xf_in = jnp.pad(x_fact, ((0, NFP - NF), (0, 0)))
    xc_in = jnp.pad(x_company, ((0, NCP - NC), (0, 0)))

    w_fc, idx_fc = _gate_idx(
        _pad_edges_f32(edge_attr_fc[:, 0]), _pad_edges_f32(edge_attr_fc[:, 1]),
        _pad_edges_i32(src_fc, 0), _pad_edges_i32(dst_fc, -1),
        p["mixer_W"], p["mixer_b"], NFP)
    w_cf, idx_cf = _gate_idx(
        _pad_edges_f32(edge_attr_cf[:, 0]), _pad_edges_f32(edge_attr_cf[:, 1]),
        _pad_edges_i32(src_cf, 0), _pad_edges_i32(dst_cf, -1),
        p["mixer_W"], p["mixer_b"], NCP)

    a_fc = _build_adj(idx_fc, w_fc, NCP, NFP)
    a_cf = _build_adj(idx_cf, w_cf, NFP, NCP)

    xf = _encoder(xf_in, p["proj_fact_W"], p["proj_fact_b"].reshape(1, H),
                  p["proj_fact_g"].reshape(1, H),
                  p["proj_fact_beta"].reshape(1, H), 1024)
    xc = _encoder(xc_in, p["proj_comp_W"], p["proj_comp_b"].reshape(1, H),
                  p["proj_comp_g"].reshape(1, H),
                  p["proj_comp_beta"].reshape(1, H), 2048)

    for l in range(2):
        xc_new = _conv_c(a_fc, xf, xc,
                         p["rel_W_%d_fc" % l], p["rel_b_%d_fc" % l].reshape(1, H),
                         p["root_W_%d_fc" % l],
                         p["post_comp_g"].reshape(1, H),
                         p["post_comp_beta"].reshape(1, H))
        xf = _conv_f(a_cf, xc, xf,
                     p["rel_W_%d_cf" % l], p["rel_b_%d_cf" % l].reshape(1, H),
                     p["root_W_%d_cf" % l],
                     p["post_fact_g"].reshape(1, H),
                     p["post_fact_beta"].reshape(1, H))
        xc = xc_new

    fb = jnp.pad(fact_batch.astype(jnp.int32), (0, NFP - NF),
                 constant_values=-1).reshape(NFP // 1024, 1, 1024)
    cb = jnp.pad(company_batch.astype(jnp.int32), (0, NCP - NC),
                 constant_values=-1).reshape(NCP // 1024, 1, 1024)
    fs, fcnt = _pool(xf, fb, 1024)
    cs, ccnt = _pool(xc, cb, 1024)

    logits = _readout(fs, fcnt, cs, ccnt,
                      p["gate_W"][:H], p["gate_W"][H:],
                      p["gate_b"].reshape(1, 1),
                      p["cls_W"], p["cls_b"].reshape(1, 1))
    return logits[:, 0]
